# baseline (speedup 1.0000x reference)
import functools

import jax
import jax.numpy as jnp
from jax import lax
from jax.experimental import pallas as pl
from jax.experimental.pallas import tpu as pltpu
from jax.experimental.pallas import tpu_sc as plsc

N_EXPERTS = 64
TOP_K = 8
N_GROUP = 8
TOPK_GROUP = 4
SCALE = 2.5
PER_GROUP = N_EXPERTS // N_GROUP

BT = 2048
TOKENS = 16384
NW = 32
TPB = TOKENS // NW
L = 16
NCHUNK = TPB // L
WPB = BT // TPB


def _scores_block(x_ref, w_ref, s_out_ref):
    logits = jax.lax.dot_general(
        w_ref[...], x_ref[...],
        dimension_numbers=(((1,), (1,)), ((), ())),
        preferred_element_type=jnp.float32,
    )
    sig = jax.nn.sigmoid(logits)
    for w in range(WPB):
        s_out_ref[w] = sig[:, w * TPB : (w + 1) * TPB]


@jax.jit
def _tc_scores(hidden_states, weight):
    s, d = hidden_states.shape
    return pl.pallas_call(
        _scores_block,
        grid=(s // BT,),
        in_specs=[
            pl.BlockSpec((BT, d), lambda i: (i, 0)),
            pl.BlockSpec((N_EXPERTS, d), lambda i: (0, 0)),
        ],
        out_specs=pl.BlockSpec((WPB, N_EXPERTS, TPB), lambda i: (i, 0, 0)),
        out_shape=jax.ShapeDtypeStruct((NW, N_EXPERTS, TPB), jnp.float32),
    )(hidden_states, weight)


def _tree_max(vs):
    while len(vs) > 1:
        vs = [jnp.maximum(vs[i], vs[i + 1]) for i in range(0, len(vs) - 1, 2)] + (
            [vs[-1]] if len(vs) % 2 else []
        )
    return vs[0]


def _tree_min(vs):
    while len(vs) > 1:
        vs = [jnp.minimum(vs[i], vs[i + 1]) for i in range(0, len(vs) - 1, 2)] + (
            [vs[-1]] if len(vs) % 2 else []
        )
    return vs[0]


def _sc_route_body(scores_hbm, biasb_hbm, wout_hbm, iout_hbm,
                   sc_v, bias_v, work_v, wout_v, iout_v):
    neg_inf = jnp.float32(-jnp.inf)
    wid = lax.axis_index("s") * 2 + lax.axis_index("c")
    base = wid * (N_EXPERTS * TPB)
    pltpu.sync_copy(scores_hbm.at[pl.ds(base, N_EXPERTS * TPB)], sc_v)
    pltpu.sync_copy(biasb_hbm, bias_v)

    def chunk(t, carry):
        col = t * L
        lanes = lax.iota(jnp.int32, L) + col

        gm = []
        for g in range(N_GROUP):
            s8 = [
                sc_v[pl.ds((g * PER_GROUP + j) * TPB + col, L)]
                + bias_v[pl.ds((g * PER_GROUP + j) * L, L)]
                for j in range(PER_GROUP)
            ]
            m1 = _tree_max(s8)
            am1 = _tree_min(
                [jnp.where(s8[j] == m1, jnp.int32(j), jnp.int32(PER_GROUP))
                 for j in range(PER_GROUP)]
            )
            m2 = _tree_max(
                [jnp.where(am1 == j, neg_inf, s8[j]) for j in range(PER_GROUP)]
            )
            gm.append(m1 + m2)
            for j in range(PER_GROUP):
                work_v[pl.ds((g * PER_GROUP + j) * TPB + col, L)] = s8[j]

        gsel = [jnp.zeros((L,), jnp.bool_) for _ in range(N_GROUP)]
        for _ in range(TOPK_GROUP):
            m = _tree_max(gm)
            amg = _tree_min(
                [jnp.where(gm[g] == m, jnp.int32(g), jnp.int32(N_GROUP))
                 for g in range(N_GROUP)]
            )
            for g in range(N_GROUP):
                hit = amg == g
                gsel[g] = gsel[g] | hit
                gm[g] = jnp.where(hit, neg_inf, gm[g])

        for g in range(N_GROUP):
            for j in range(PER_GROUP):
                off = (g * PER_GROUP + j) * TPB + col
                v = work_v[pl.ds(off, L)]
                work_v[pl.ds(off, L)] = jnp.where(gsel[g], v, 0.0)

        wvs = []
        for k in range(TOP_K):
            v = [work_v[pl.ds(e * TPB + col, L)] for e in range(N_EXPERTS)]
            m = _tree_max(v)
            am = _tree_min(
                [jnp.where(v[e] == m, jnp.int32(e), jnp.int32(N_EXPERTS))
                 for e in range(N_EXPERTS)]
            )
            wv = plsc.load_gather(sc_v, [am * TPB + lanes])
            wvs.append(wv)
            iout_v[pl.ds(k * TPB + col, L)] = am
            plsc.store_scatter(work_v, [am * TPB + lanes], jnp.full((L,), neg_inf))

        denom = wvs[0]
        for wv in wvs[1:]:
            denom = denom + wv
        inv = SCALE / (denom + 1e-20)
        for k in range(TOP_K):
            wout_v[pl.ds(k * TPB + col, L)] = wvs[k] * inv
        return carry

    lax.fori_loop(0, NCHUNK, chunk, 0)

    obase = wid * (TOP_K * TPB)
    pltpu.sync_copy(wout_v, wout_hbm.at[pl.ds(obase, TOP_K * TPB)])
    pltpu.sync_copy(iout_v, iout_hbm.at[pl.ds(obase, TOP_K * TPB)])


@jax.jit
def _sc_route(scores_flat, bias_flat):
    mesh = plsc.VectorSubcoreMesh(core_axis_name="c", subcore_axis_name="s")
    return pl.kernel(
        _sc_route_body,
        mesh=mesh,
        out_type=[
            jax.ShapeDtypeStruct((NW * TOP_K * TPB,), jnp.float32),
            jax.ShapeDtypeStruct((NW * TOP_K * TPB,), jnp.int32),
        ],
        scratch_types=[
            pltpu.VMEM((N_EXPERTS * TPB,), jnp.float32),
            pltpu.VMEM((N_EXPERTS * L,), jnp.float32),
            pltpu.VMEM((N_EXPERTS * TPB,), jnp.float32),
            pltpu.VMEM((TOP_K * TPB,), jnp.float32),
            pltpu.VMEM((TOP_K * TPB,), jnp.int32),
        ],
        compiler_params=pltpu.CompilerParams(needs_layout_passes=False),
    )(scores_flat, bias_flat)


def kernel(hidden_states, weight, e_score_correction_bias):
    scores = _tc_scores(hidden_states, weight)
    bias_flat = jnp.broadcast_to(
        e_score_correction_bias.reshape(N_EXPERTS, 1), (N_EXPERTS, L)
    ).reshape(-1)
    w_flat, i_flat = _sc_route(scores.reshape(-1), bias_flat)
    topk_weight = (
        w_flat.reshape(NW, TOP_K, TPB).transpose(0, 2, 1).reshape(TOKENS, TOP_K)
    )
    topk_idx = (
        i_flat.reshape(NW, TOP_K, TPB).transpose(0, 2, 1).reshape(TOKENS, TOP_K)
    )
    return (topk_weight, topk_idx)
